# baseline (speedup 1.0000x reference)
import functools

import jax
import jax.numpy as jnp
from jax import lax
from jax.experimental import pallas as pl
from jax.experimental.pallas import tpu as pltpu
from jax.experimental.pallas import tpu_sc as plsc

NC = 2
NS = 16
L = 16
NW = NC * NS
NB = 3


def _fast_rsqrt(d):
    i = lax.bitcast_convert_type(d, jnp.int32)
    i = jnp.int32(0x5F3759DF) - jnp.right_shift(i, 1)
    y = lax.bitcast_convert_type(i, jnp.float32)
    h = d * 0.5
    for _ in range(3):
        y = y * (1.5 - h * y * y)
    return y


def _ring(nchunk, fetch, proc, wait_scatter):
    def step(c, j, last):
        if not last:
            nj = (j + 1) % NB

            @pl.when(c >= NB - 1)
            def _():
                wait_scatter(nj)

            fetch(c + 1, nj)
        proc(c, j)

    fetch(0, 0)
    T = (nchunk - 1) // NB

    def body(t, _):
        for j in range(NB):
            step(NB * t + j, j, False)
        return 0
    lax.fori_loop(0, T, body, 0)
    for c in range(NB * T, nchunk):
        step(c, c % NB, c == nchunk - 1)
    for c in range(nchunk - NB, nchunk):
        wait_scatter(c % NB)


def _sc_body(E, NP, D, CH, SL, EW0, EW1, xp_ref, esrc_ref, edst_ref, w_ref,
             rso_ref, rsi_ref, rso2_ref, aggp_ref,
             src_v, coef_v, ones_v, hist_v, rs_v,
             wch_a, wch_b, wch_c, crs_a, crs_b, crs_c,
             dch_a, dch_b, dch_c, rows_a, rows_b, rows_c,
             agg_sh, rso_sh, hdst_sh,
             gsem_a, gsem_b, gsem_c, msem_a, msem_b, msem_c,
             ssem_a, ssem_b, ssem_c):
    cid = lax.axis_index("c")
    sid = lax.axis_index("s")
    EH = E // NS
    NH = EH // CH
    E0TOT = NS * EW0
    bufs = (rows_a, rows_b, rows_c)
    wchs = (wch_a, wch_b, wch_c)
    crss = (crs_a, crs_b, crs_c)
    dchs = (dch_a, dch_b, dch_c)
    gsems = (gsem_a, gsem_b, gsem_c)
    msems = (msem_a, msem_b, msem_c)
    ssems = (ssem_a, ssem_b, ssem_c)

    def fill_ones(i, _):
        ones_v[pl.ds(i * L, L)] = jnp.full((L,), 1.0, jnp.float32)
        return 0
    lax.fori_loop(0, CH // L, fill_ones, 0)

    def fill_zero(i, _):
        rs_v[pl.ds(i * L, L)] = jnp.zeros((L,), jnp.float32)
        return 0
    lax.fori_loop(0, SL // L, fill_zero, 0)

    def zrow(i, _):
        rows_a[i // (D // L), pl.ds((i % (D // L)) * L, L)] = (
            jnp.zeros((L,), jnp.float32))
        return 0
    lax.fori_loop(0, CH * (D // L), zrow, 0)
    for k in range(SL // CH):
        pltpu.sync_copy(rows_a, agg_sh.at[pl.ds(sid * SL + k * CH, CH)])
    pltpu.sync_copy(rs_v, rso_sh.at[pl.ds(sid * SL, SL)])
    pltpu.sync_copy(rs_v, hdst_sh.at[pl.ds(sid * SL, SL)])

    @pl.when(cid == 0)
    def _():
        pltpu.sync_copy(esrc_ref.at[pl.ds(sid * EW0, EW0)],
                        src_v.at[pl.ds(0, EW0)])

    @pl.when(cid == 1)
    def _():
        pltpu.sync_copy(esrc_ref.at[pl.ds(E0TOT + sid * EW1, EW1)],
                        src_v.at[pl.ds(0, EW1)])

    plsc.subcore_barrier()

    def hist_loop(idx_hbm, target_sh):
        def fetch(c, j):
            pltpu.async_copy(idx_hbm.at[pl.ds(sid * EH + c * CH, CH)],
                             dchs[j], msems[j])

        def proc(c, j):
            pltpu.make_async_copy(
                idx_hbm.at[pl.ds(sid * EH + c * CH, CH)], dchs[j],
                msems[j]).wait()
            pltpu.async_copy(ones_v, target_sh.at[dchs[j]], ssems[j],
                             add=True)

        def wait_scatter(j):
            pltpu.make_async_copy(ones_v, target_sh.at[dchs[j]],
                                  ssems[j]).wait()

        _ring(NH, fetch, proc, wait_scatter)

    @pl.when(cid == 0)
    def _():
        hist_loop(esrc_ref, rso_sh)

    @pl.when(cid == 1)
    def _():
        hist_loop(esrc_ref, rso_sh)
        hist_loop(edst_ref, hdst_sh)

    plsc.subcore_barrier()

    pltpu.sync_copy(rso_sh.at[pl.ds(sid * SL, SL)], hist_v)

    def rsq(g, _):
        d = hist_v[pl.ds(g * L, L)] + 1.0
        rs_v[pl.ds(g * L, L)] = _fast_rsqrt(d)
        return 0
    lax.fori_loop(0, SL // L, rsq, 0)

    @pl.when(cid == 0)
    def _():
        pltpu.sync_copy(rs_v, rso_ref.at[pl.ds(sid * SL, SL)])

    @pl.when(cid == 1)
    def _():
        pltpu.sync_copy(rs_v, rso2_ref.at[pl.ds(sid * SL, SL)])
        pltpu.sync_copy(hdst_sh.at[pl.ds(sid * SL, SL)], hist_v)
        lax.fori_loop(0, SL // L, rsq, 0)
        pltpu.sync_copy(rs_v, rsi_ref.at[pl.ds(sid * SL, SL)])

    plsc.subcore_barrier()

    def agg_loop(base, nch, rss_ref):
        def fetch(c, j):
            idx = src_v.at[pl.ds(c * CH, CH)]
            pltpu.async_copy(xp_ref.at[idx], bufs[j], gsems[j])
            pltpu.async_copy(w_ref.at[pl.ds(base + c * CH, CH)], wchs[j],
                             msems[j])
            pltpu.async_copy(edst_ref.at[pl.ds(base + c * CH, CH)], dchs[j],
                             msems[j])
            pltpu.async_copy(rss_ref.at[idx], crss[j], msems[j])

        def proc(c, j):
            buf = bufs[j]
            idx = src_v.at[pl.ds(c * CH, CH)]
            pltpu.make_async_copy(xp_ref.at[idx], buf, gsems[j]).wait()
            pltpu.make_async_copy(
                w_ref.at[pl.ds(base + c * CH, CH)], wchs[j], msems[j]).wait()
            pltpu.make_async_copy(
                edst_ref.at[pl.ds(base + c * CH, CH)], dchs[j],
                msems[j]).wait()
            pltpu.make_async_copy(rss_ref.at[idx], crss[j], msems[j]).wait()

            for g in range(CH // L):
                coef_v[pl.ds(g * L, L)] = (
                    wchs[j][pl.ds(g * L, L)] * crss[j][pl.ds(g * L, L)])

            U = 8

            def scale(ru, _):
                r0 = ru * U
                cbs = [plsc.load_gather(coef_v,
                                        [jnp.full((L,), r0 + k, jnp.int32)])
                       for k in range(U)]
                for k in range(U):
                    for jj in range(D // L):
                        buf[r0 + k, pl.ds(jj * L, L)] = (
                            buf[r0 + k, pl.ds(jj * L, L)] * cbs[k])
                return 0
            lax.fori_loop(0, CH // U, scale, 0)

            pltpu.async_copy(buf, agg_sh.at[dchs[j]], ssems[j], add=True)

        def wait_scatter(j):
            pltpu.make_async_copy(bufs[j], agg_sh.at[dchs[j]],
                                  ssems[j]).wait()

        _ring(nch, fetch, proc, wait_scatter)

    @pl.when(cid == 0)
    def _():
        agg_loop(sid * EW0, EW0 // CH, rso_ref)

    @pl.when(cid == 1)
    def _():
        agg_loop(E0TOT + sid * EW1, EW1 // CH, rso2_ref)

    plsc.subcore_barrier()
    pltpu.sync_copy(agg_sh.at[pl.ds(sid * SL, SL)],
                    aggp_ref.at[cid, pl.ds(sid * SL, SL)])


def _mm_body(p_ref, xp_ref, rso_ref, rsi_ref, w_ref, b_ref, o_ref):
    p = p_ref[...]
    agg = p[0] + p[1] + xp_ref[...] * rso_ref[...]
    acc = agg * rsi_ref[...]
    o_ref[...] = (jnp.dot(acc, w_ref[...], preferred_element_type=jnp.float32)
                  + b_ref[...])


@jax.jit
def kernel(x, edge_index, edge_weight, W, b):
    N, D = x.shape
    E = edge_index.shape[1]
    NP = ((N + 639) // 640) * 640
    SL = NP // NS
    CH = 80
    EVEN = E // NW
    EW0 = EVEN + 8 * CH
    EW1 = EVEN - 8 * CH

    esrc = edge_index[0]
    edst = edge_index[1]

    mesh = plsc.VectorSubcoreMesh(core_axis_name="c", subcore_axis_name="s")
    sc_params = pltpu.CompilerParams(needs_layout_passes=False)

    sc_k = pl.kernel(
        functools.partial(_sc_body, E, NP, D, CH, SL, EW0, EW1),
        out_type=[
            jax.ShapeDtypeStruct((NP,), jnp.float32),
            jax.ShapeDtypeStruct((NP,), jnp.float32),
            jax.ShapeDtypeStruct((NP,), jnp.float32),
            jax.ShapeDtypeStruct((NC, NP, D), jnp.float32),
        ],
        mesh=mesh,
        scratch_types=(
            [pltpu.VMEM((EW0,), jnp.int32),
             pltpu.VMEM((CH,), jnp.float32),
             pltpu.VMEM((CH,), jnp.float32),
             pltpu.VMEM((SL,), jnp.float32),
             pltpu.VMEM((SL,), jnp.float32)]
            + [pltpu.VMEM((CH,), jnp.float32)] * 6
            + [pltpu.VMEM((CH,), jnp.int32)] * 3
            + [pltpu.VMEM((CH, D), jnp.float32)] * 3
            + [pltpu.VMEM_SHARED((NP, D), jnp.float32),
               pltpu.VMEM_SHARED((NP,), jnp.float32),
               pltpu.VMEM_SHARED((NP,), jnp.float32)]
            + [pltpu.SemaphoreType.DMA] * 9
        ),
        compiler_params=sc_params,
    )
    rso, rsi, _, aggp = sc_k(x, esrc, edst, edge_weight)

    BR = 1024
    out = pl.pallas_call(
        _mm_body,
        grid=(NP // BR,),
        in_specs=[
            pl.BlockSpec((NC, BR, D), lambda i: (0, i, 0)),
            pl.BlockSpec((BR, D), lambda i: (i, 0)),
            pl.BlockSpec((BR, 1), lambda i: (i, 0)),
            pl.BlockSpec((BR, 1), lambda i: (i, 0)),
            pl.BlockSpec((D, D), lambda i: (0, 0)),
            pl.BlockSpec((1, D), lambda i: (0, 0)),
        ],
        out_specs=pl.BlockSpec((BR, D), lambda i: (i, 0)),
        out_shape=jax.ShapeDtypeStruct((N, D), jnp.float32),
    )(aggp, x, rso.reshape(NP, 1), rsi.reshape(NP, 1), W, b.reshape(1, D))

    return out


# speedup vs baseline: 1.4998x; 1.4998x over previous
import functools

import jax
import jax.numpy as jnp
from jax import lax
from jax.experimental import pallas as pl
from jax.experimental.pallas import tpu as pltpu
from jax.experimental.pallas import tpu_sc as plsc

NC = 2
NS = 16
L = 16
NW = NC * NS


def _fast_rsqrt(d):
    i = lax.bitcast_convert_type(d, jnp.int32)
    i = jnp.int32(0x5F3759DF) - jnp.right_shift(i, 1)
    y = lax.bitcast_convert_type(i, jnp.float32)
    h = d * 0.5
    for _ in range(3):
        y = y * (1.5 - h * y * y)
    return y


def _deg_body(E, NP, CH, SL, src_ref, dst_ref, rso_ref, rsi_ref,
              idx2_v, ones_v, hist_v, rs_v, deg_sh, ssem):
    cid = lax.axis_index("c")
    sid = lax.axis_index("s")
    EC = E // NS
    NCH = EC // CH
    K = 10

    def fill_ones(i, _):
        ones_v[pl.ds(i * L, L)] = jnp.full((L,), 1.0, jnp.float32)
        return 0
    lax.fori_loop(0, CH // L, fill_ones, 0)

    def fill_zero(i, _):
        rs_v[pl.ds(i * L, L)] = jnp.zeros((L,), jnp.float32)
        return 0
    lax.fori_loop(0, SL // L, fill_zero, 0)

    @pl.when(cid == 0)
    def _():
        pltpu.sync_copy(src_ref.at[sid], idx2_v)

    @pl.when(cid == 1)
    def _():
        pltpu.sync_copy(dst_ref.at[sid], idx2_v)

    pltpu.sync_copy(rs_v, deg_sh.at[pl.ds(sid * SL, SL)])
    plsc.subcore_barrier()

    def fire_drain(t, _):
        for j in range(K):
            pltpu.async_copy(ones_v, deg_sh.at[idx2_v.at[t * K + j]], ssem,
                             add=True)
        for j in range(K):
            pltpu.make_async_copy(ones_v, deg_sh.at[idx2_v.at[t * K + j]],
                                  ssem).wait()
        return 0
    lax.fori_loop(0, NCH // K, fire_drain, 0)
    plsc.subcore_barrier()

    pltpu.sync_copy(deg_sh.at[pl.ds(sid * SL, SL)], hist_v)

    def rsq(g, _):
        d = hist_v[pl.ds(g * L, L)] + 1.0
        rs_v[pl.ds(g * L, L)] = _fast_rsqrt(d)
        return 0
    lax.fori_loop(0, SL // L, rsq, 0)

    @pl.when(cid == 0)
    def _():
        pltpu.sync_copy(rs_v, rso_ref.at[pl.ds(sid * SL, SL)])

    @pl.when(cid == 1)
    def _():
        pltpu.sync_copy(rs_v, rsi_ref.at[pl.ds(sid * SL, SL)])


def _agg_body(E, NP, D, CH, SL, xp_ref, esrc_ref, edst_ref, w_ref, rso_ref,
              aggp_ref,
              src_v, coef_v, wch_a, wch_b, wch_c, crs_a, crs_b, crs_c,
              dch_a, dch_b, dch_c, rows_a, rows_b, rows_c, agg_sh,
              gsem_a, gsem_b, gsem_c, msem_a, msem_b, msem_c,
              ssem_a, ssem_b, ssem_c):
    cid = lax.axis_index("c")
    sid = lax.axis_index("s")
    wid = cid * NS + sid
    EW = E // NW
    NCH = EW // CH
    NB = 3
    bufs = (rows_a, rows_b, rows_c)
    wchs = (wch_a, wch_b, wch_c)
    crss = (crs_a, crs_b, crs_c)
    dchs = (dch_a, dch_b, dch_c)
    gsems = (gsem_a, gsem_b, gsem_c)
    msems = (msem_a, msem_b, msem_c)
    ssems = (ssem_a, ssem_b, ssem_c)

    pltpu.sync_copy(esrc_ref.at[pl.ds(wid * EW, EW)], src_v)

    def zrow(i, _):
        rows_a[i // (D // L), pl.ds((i % (D // L)) * L, L)] = (
            jnp.zeros((L,), jnp.float32))
        return 0
    lax.fori_loop(0, CH * (D // L), zrow, 0)
    for k in range(SL // CH):
        pltpu.sync_copy(rows_a, agg_sh.at[pl.ds(sid * SL + k * CH, CH)])
    plsc.subcore_barrier()

    def fetch(c, b):
        idx = src_v.at[pl.ds(c * CH, CH)]
        pltpu.async_copy(xp_ref.at[idx], bufs[b], gsems[b])
        pltpu.async_copy(w_ref.at[pl.ds(wid * EW + c * CH, CH)], wchs[b],
                         msems[b])
        pltpu.async_copy(edst_ref.at[pl.ds(wid * EW + c * CH, CH)], dchs[b],
                         msems[b])
        pltpu.async_copy(rso_ref.at[idx], crss[b], msems[b])

    def wait_scatter(b):
        pltpu.make_async_copy(bufs[b], agg_sh.at[dchs[b]], ssems[b]).wait()

    def process(c, b, last=False):
        buf = bufs[b]
        idx = src_v.at[pl.ds(c * CH, CH)]

        if not last:
            nb = (b + 1) % NB

            @pl.when(c >= NB - 1)
            def _():
                wait_scatter(nb)

            fetch(c + 1, nb)

        pltpu.make_async_copy(xp_ref.at[idx], buf, gsems[b]).wait()
        pltpu.make_async_copy(
            w_ref.at[pl.ds(wid * EW + c * CH, CH)], wchs[b], msems[b]).wait()
        pltpu.make_async_copy(
            edst_ref.at[pl.ds(wid * EW + c * CH, CH)], dchs[b],
            msems[b]).wait()
        pltpu.make_async_copy(rso_ref.at[idx], crss[b], msems[b]).wait()

        for g in range(CH // L):
            coef_v[pl.ds(g * L, L)] = (
                wchs[b][pl.ds(g * L, L)] * crss[b][pl.ds(g * L, L)])

        U = 8

        def scale(ru, _):
            r0 = ru * U
            cbs = [plsc.load_gather(coef_v, [jnp.full((L,), r0 + k, jnp.int32)])
                   for k in range(U)]
            for k in range(U):
                for j in range(D // L):
                    buf[r0 + k, pl.ds(j * L, L)] = (
                        buf[r0 + k, pl.ds(j * L, L)] * cbs[k])
            return 0
        lax.fori_loop(0, CH // U, scale, 0)

        pltpu.async_copy(buf, agg_sh.at[dchs[b]], ssems[b], add=True)

    fetch(0, 0)

    def triple(t, _):
        for j in range(NB):
            process(NB * t + j, j)
        return 0
    lax.fori_loop(0, NCH // NB, triple, 0)
    base = (NCH // NB) * NB
    for c in range(base, NCH):
        process(c, c % NB, last=(c == NCH - 1))
    for c in range(max(base, NCH - NB + 1) - 1, NCH):
        wait_scatter(c % NB)

    plsc.subcore_barrier()
    pltpu.sync_copy(agg_sh.at[pl.ds(sid * SL, SL)],
                    aggp_ref.at[cid, pl.ds(sid * SL, SL)])


def _mm_body(p_ref, xp_ref, rso_ref, rsi_ref, w_ref, b_ref, o_ref):
    p = p_ref[...]
    agg = p[0] + p[1] + xp_ref[...] * rso_ref[...]
    acc = agg * rsi_ref[...]
    o_ref[...] = (jnp.dot(acc, w_ref[...], preferred_element_type=jnp.float32)
                  + b_ref[...])


@jax.jit
def kernel(x, edge_index, edge_weight, W, b):
    N, D = x.shape
    E = edge_index.shape[1]
    NP = ((N + 639) // 640) * 640
    SL = NP // NS
    CH = 80

    esrc = edge_index[0]
    edst = edge_index[1]
    esrc3 = esrc.reshape(NS, (E // NS) // CH, CH)
    edst3d = edst.reshape(NS, (E // NS) // CH, CH)

    mesh = plsc.VectorSubcoreMesh(core_axis_name="c", subcore_axis_name="s")
    sc_params = pltpu.CompilerParams(needs_layout_passes=False)

    deg_k = pl.kernel(
        functools.partial(_deg_body, E, NP, CH, SL),
        out_type=[
            jax.ShapeDtypeStruct((NP,), jnp.float32),
            jax.ShapeDtypeStruct((NP,), jnp.float32),
        ],
        mesh=mesh,
        scratch_types=[
            pltpu.VMEM(((E // NS) // CH, CH), jnp.int32),
            pltpu.VMEM((CH,), jnp.float32),
            pltpu.VMEM((SL,), jnp.float32),
            pltpu.VMEM((SL,), jnp.float32),
            pltpu.VMEM_SHARED((NP,), jnp.float32),
            pltpu.SemaphoreType.DMA,
        ],
        compiler_params=sc_params,
    )
    rso, rsi = deg_k(esrc3, edst3d)

    agg_k = pl.kernel(
        functools.partial(_agg_body, E, NP, D, CH, SL),
        out_type=jax.ShapeDtypeStruct((NC, NP, D), jnp.float32),
        mesh=mesh,
        scratch_types=(
            [pltpu.VMEM((E // NW,), jnp.int32),
             pltpu.VMEM((CH,), jnp.float32)]
            + [pltpu.VMEM((CH,), jnp.float32)] * 6
            + [pltpu.VMEM((CH,), jnp.int32)] * 3
            + [pltpu.VMEM((CH, D), jnp.float32)] * 3
            + [pltpu.VMEM_SHARED((NP, D), jnp.float32)]
            + [pltpu.SemaphoreType.DMA] * 9
        ),
        compiler_params=sc_params,
    )
    aggp = agg_k(x, esrc, edst, edge_weight, rso)

    BR = 1024
    out = pl.pallas_call(
        _mm_body,
        grid=(NP // BR,),
        in_specs=[
            pl.BlockSpec((NC, BR, D), lambda i: (0, i, 0)),
            pl.BlockSpec((BR, D), lambda i: (i, 0)),
            pl.BlockSpec((BR, 1), lambda i: (i, 0)),
            pl.BlockSpec((BR, 1), lambda i: (i, 0)),
            pl.BlockSpec((D, D), lambda i: (0, 0)),
            pl.BlockSpec((1, D), lambda i: (0, 0)),
        ],
        out_specs=pl.BlockSpec((BR, D), lambda i: (i, 0)),
        out_shape=jax.ShapeDtypeStruct((N, D), jnp.float32),
    )(aggp, x, rso.reshape(NP, 1), rsi.reshape(NP, 1), W, b.reshape(1, D))

    return out
